# Initial kernel scaffold; baseline (speedup 1.0000x reference)
import jax
import jax.numpy as jnp
from jax.experimental import pallas as pl


def kernel(x, stdn, rbf_data, conv_weights, scale_f, alpha_prox, rbf_weights, rbf_centers):
    raise NotImplementedError("write your pallas kernel here")


# speedup vs baseline: 1.5302x; 1.5302x over previous
import functools

import jax
import jax.numpy as jnp
from jax.experimental import pallas as pl
from jax.experimental.pallas import tpu as pltpu

RBF_CHUNK = 256
_VMEM_LIMIT = 48 * 1024 * 1024


def _round_up(a, b):
    return -(-a // b) * b


def _conv1_rbf_kernel(scal_ref, cent_ref, w_ref, rbfw_ref, x_ref, xh_ref,
                      o_ref, xbuf_ref, slab_ref, v_ref, *, kh, kw, wrow, chunk):
    tl = o_ref.shape[2]
    kk = kh * kw
    xbuf_ref[0, :tl] = x_ref[0, 0]
    xbuf_ref[0, tl:] = xh_ref[0, 0]
    for t in range(kk):
        i, j = divmod(t, kw)
        s = i * wrow + j
        slab_ref[t, :] = xbuf_ref[0, s:s + tl]
    slab_ref[kk:, :] = jnp.zeros((slab_ref.shape[0] - kk, tl), jnp.float32)
    v_ref[...] = jnp.dot(w_ref[...], slab_ref[...],
                         preferred_element_type=jnp.float32)
    sp = scal_ref[0]
    lb_s = scal_ref[1]
    ub_s = scal_ref[2]
    n_out = o_ref.shape[1]
    n_cent = cent_ref.shape[0]
    for c0 in range(0, tl, chunk):
        cw = min(chunk, tl - c0)
        u = jnp.clip(v_ref[:, c0:c0 + cw] * sp, lb_s, ub_s)
        res = jnp.zeros((n_out, cw), jnp.float32)
        for m in range(n_cent):
            d = u - cent_ref[m]
            res = res + rbfw_ref[:, m:m + 1] * jnp.exp(-(d * d))
        o_ref[0, :, c0:c0 + cw] = res


def _convT_kernel(w_ref, x_ref, xh_ref, o_ref, xbuf_ref, g_ref, *, kh, kw, wrow):
    tl = o_ref.shape[2]
    xbuf_ref[:, :tl] = x_ref[0]
    xbuf_ref[:, tl:] = xh_ref[0]
    g_ref[...] = jnp.dot(w_ref[...], xbuf_ref[...],
                         preferred_element_type=jnp.float32)
    acc = jnp.zeros((1, tl), jnp.float32)
    for t in range(kh * kw):
        i, j = divmod(t, kw)
        s = i * wrow + j
        acc = acc + g_ref[t:t + 1, s:s + tl]
    o_ref[0] = acc


def _residual_kernel(stdn_ref, x_ref, z_ref, o_ref):
    n = pl.program_id(0)
    s2 = stdn_ref[n] * stdn_ref[n]
    o_ref[...] = x_ref[...] - s2 * z_ref[...]


def _flat_tiling(lout, halo):
    halo_pad = _round_up(halo, 128)
    tl = 2 * halo_pad
    nt = pl.cdiv(lout, tl)
    lf = nt * tl + halo_pad
    return tl, nt, lf, halo_pad


def _conv1_rbf(xpad, w, rbf_w, centers, prec, lb, ub, kh, kw):
    N, _, Hi, Wi = xpad.shape
    F = w.shape[0]
    KK = kh * kw
    Ho = Hi - kh + 1
    lout = Ho * Wi
    halo = (kh - 1) * Wi + (kw - 1)
    tl, nt, lf, halo_pad = _flat_tiling(lout, halo)

    xflat = jnp.pad(xpad.reshape(N, 1, Hi * Wi), ((0, 0), (0, 0), (0, lf - Hi * Wi)))
    sp = jnp.sqrt(jnp.asarray(prec, jnp.float32))
    scal = jnp.stack([sp, sp * jnp.asarray(lb, jnp.float32),
                      sp * jnp.asarray(ub, jnp.float32)])
    kk_pad = _round_up(KK, 8)
    w_stack = jnp.pad(w.transpose(0, 2, 3, 1).reshape(F, KK), ((0, 0), (0, kk_pad - KK)))

    ratio = tl // halo_pad
    x_spec = pl.BlockSpec((1, 1, tl), lambda n, t: (n, 0, t))
    halo_spec = pl.BlockSpec((1, 1, halo_pad), lambda n, t: (n, 0, (t + 1) * ratio))
    smem = pl.BlockSpec(memory_space=pltpu.MemorySpace.SMEM)
    out = pl.pallas_call(
        functools.partial(_conv1_rbf_kernel, kh=kh, kw=kw, wrow=Wi,
                          chunk=min(RBF_CHUNK, tl)),
        out_shape=jax.ShapeDtypeStruct((N, F, nt * tl), jnp.float32),
        grid=(N, nt),
        in_specs=[smem, smem,
                  pl.BlockSpec((F, kk_pad), lambda n, t: (0, 0)),
                  pl.BlockSpec((F, centers.shape[0]), lambda n, t: (0, 0)),
                  x_spec, halo_spec],
        out_specs=pl.BlockSpec((1, F, tl), lambda n, t: (n, 0, t)),
        scratch_shapes=[pltpu.VMEM((1, tl + halo_pad), jnp.float32),
                        pltpu.VMEM((kk_pad, tl), jnp.float32),
                        pltpu.VMEM((F, tl), jnp.float32)],
        compiler_params=pltpu.CompilerParams(
            dimension_semantics=("parallel", "parallel"),
            vmem_limit_bytes=_VMEM_LIMIT),
    )(scal, sp * centers.astype(jnp.float32), w_stack,
      rbf_w.astype(jnp.float32), xflat, xflat)
    return out[:, :, :lout].reshape(N, F, Ho, Wi)[:, :, :, :Wi - kw + 1]


def _convT(v, w, kh, kw):
    N, F, H, W = v.shape
    Hi, Wi = H + 2 * (kh - 1), W + 2 * (kw - 1)
    Ho, Wo = Hi - kh + 1, Wi - kw + 1
    KK = kh * kw
    lout = Ho * Wi
    halo = (kh - 1) * Wi + (kw - 1)
    tl, nt, lf, halo_pad = _flat_tiling(lout, halo)

    vzp = jnp.pad(v, ((0, 0), (0, 0), (kh - 1, kh - 1), (kw - 1, kw - 1)))
    vflat = jnp.pad(vzp.reshape(N, F, Hi * Wi), ((0, 0), (0, 0), (0, lf - Hi * Wi)))

    wt = jnp.flip(w, (-2, -1)).reshape(F, KK).transpose(1, 0)
    kk_pad = _round_up(KK, 8)
    wt = jnp.pad(wt, ((0, kk_pad - KK), (0, 0)))

    ratio = tl // halo_pad
    x_spec = pl.BlockSpec((1, F, tl), lambda n, t: (n, 0, t))
    halo_spec = pl.BlockSpec((1, F, halo_pad), lambda n, t: (n, 0, (t + 1) * ratio))
    out = pl.pallas_call(
        functools.partial(_convT_kernel, kh=kh, kw=kw, wrow=Wi),
        out_shape=jax.ShapeDtypeStruct((N, 1, nt * tl), jnp.float32),
        grid=(N, nt),
        in_specs=[pl.BlockSpec((kk_pad, F), lambda n, t: (0, 0)),
                  x_spec, halo_spec],
        out_specs=pl.BlockSpec((1, 1, tl), lambda n, t: (n, 0, t)),
        scratch_shapes=[pltpu.VMEM((F, tl + halo_pad), jnp.float32),
                        pltpu.VMEM((kk_pad, tl + halo_pad), jnp.float32)],
        compiler_params=pltpu.CompilerParams(
            dimension_semantics=("parallel", "parallel"),
            vmem_limit_bytes=_VMEM_LIMIT),
    )(wt, vflat, vflat)
    return out[:, :, :lout].reshape(N, 1, Ho, Wi)[:, :, :, :Wo]


def _sym_pad_adjoint(z, pad):
    pt, pb, pL, pr = pad
    H2, W2 = z.shape[2], z.shape[3]
    core = z[:, :, pt:H2 - pb, :]
    if pt:
        core = core.at[:, :, :pt, :].add(z[:, :, :pt, :][:, :, ::-1, :])
    if pb:
        core = core.at[:, :, -pb:, :].add(z[:, :, H2 - pb:, :][:, :, ::-1, :])
    z = core
    core = z[:, :, :, pL:W2 - pr]
    if pL:
        core = core.at[:, :, :, :pL].add(z[:, :, :, :pL][:, :, :, ::-1])
    if pr:
        core = core.at[:, :, :, -pr:].add(z[:, :, :, W2 - pr:][:, :, :, ::-1])
    return core


def _residual(x, z, stdn, lanes=512, row_block=128):
    N, C, H, W = x.shape
    chw = C * H * W
    lanes = min(lanes, chw)
    rows = chw // lanes
    row_block = min(row_block, rows)
    x3 = x.reshape(N, rows, lanes)
    z3 = z.reshape(N, rows, lanes)
    blk = pl.BlockSpec((1, row_block, lanes), lambda n, r: (n, r, 0))
    smem = pl.BlockSpec(memory_space=pltpu.MemorySpace.SMEM)
    out = pl.pallas_call(
        _residual_kernel,
        out_shape=jax.ShapeDtypeStruct((N, rows, lanes), jnp.float32),
        grid=(N, rows // row_block),
        in_specs=[smem, blk, blk],
        out_specs=blk,
        compiler_params=pltpu.CompilerParams(
            dimension_semantics=("parallel", "parallel"),
            vmem_limit_bytes=_VMEM_LIMIT),
    )(stdn.astype(jnp.float32), x3, z3)
    return out.reshape(N, C, H, W)


def kernel(x, stdn, rbf_data, conv_weights, scale_f, alpha_prox, rbf_weights, rbf_centers):
    del rbf_data, alpha_prox
    kh = kw = 5
    M = rbf_centers.shape[0]
    lb, ub = -100.0, 100.0
    delta = (ub - lb) / (M - 1)
    prec = float(1.0 / (2.0 * delta * delta))
    pad = (kh // 2, kh // 2, kw // 2, kw // 2)

    w = conv_weights - conv_weights.mean(axis=(1, 2, 3), keepdims=True)
    nrm = jnp.sqrt(jnp.sum(w * w, axis=(1, 2, 3), keepdims=True)) + 1e-12
    w = (w / nrm) * scale_f[:, None, None, None]
    w = w.astype(jnp.float32)

    xpad = jnp.pad(x, ((0, 0), (0, 0), (pad[0], pad[1]), (pad[2], pad[3])),
                   mode="symmetric")
    v = _conv1_rbf(xpad, w, rbf_weights, rbf_centers, prec, lb, ub, kh, kw)
    zfull = _convT(v, w, kh, kw)
    z = _sym_pad_adjoint(zfull, pad)
    return _residual(x, z, stdn)


# speedup vs baseline: 1.6625x; 1.0864x over previous
import functools

import jax
import jax.numpy as jnp
from jax.experimental import pallas as pl
from jax.experimental.pallas import tpu as pltpu

RBF_CHUNK = 256
_VMEM_LIMIT = 48 * 1024 * 1024


def _round_up(a, b):
    return -(-a // b) * b


def _conv1_rbf_kernel(scal_ref, cent_ref, w_ref, rbfw_ref, x_ref, xh_ref,
                      o_ref, xbuf_ref, slab_ref, v_ref, *, kh, kw, wrow, chunk):
    tl = o_ref.shape[2]
    kk = kh * kw
    xbuf_ref[0, :tl] = x_ref[0, 0]
    xbuf_ref[0, tl:] = xh_ref[0, 0]
    for t in range(kk):
        i, j = divmod(t, kw)
        s = i * wrow + j
        slab_ref[t, :] = xbuf_ref[0, s:s + tl]
    slab_ref[kk:, :] = jnp.zeros((slab_ref.shape[0] - kk, tl), jnp.float32)
    v_ref[...] = jnp.dot(w_ref[...], slab_ref[...],
                         preferred_element_type=jnp.float32)
    sp = scal_ref[0]
    lb_s = scal_ref[1]
    ub_s = scal_ref[2]
    n_out = o_ref.shape[1]
    n_cent = cent_ref.shape[0]
    for c0 in range(0, tl, chunk):
        cw = min(chunk, tl - c0)
        u = jnp.clip(v_ref[:, c0:c0 + cw] * sp, lb_s, ub_s)
        res = jnp.zeros((n_out, cw), jnp.float32)
        for m in range(n_cent):
            d = u - cent_ref[m]
            res = res + rbfw_ref[:, m:m + 1] * jnp.exp(-(d * d))
        o_ref[0, :, c0:c0 + cw] = res.astype(o_ref.dtype)


def _convT_kernel(w_ref, x_ref, xh_ref, o_ref, xbuf_ref, g_ref, *, kh, kw, wrow):
    tl = o_ref.shape[2]
    xbuf_ref[:, :tl] = x_ref[0]
    xbuf_ref[:, tl:] = xh_ref[0]
    g_ref[...] = jnp.dot(w_ref[...], xbuf_ref[...],
                         preferred_element_type=jnp.float32)
    acc = jnp.zeros((1, tl), jnp.float32)
    for t in range(kh * kw):
        i, j = divmod(t, kw)
        s = i * wrow + j
        acc = acc + g_ref[t:t + 1, s:s + tl]
    o_ref[0] = acc


def _residual_kernel(stdn_ref, x_ref, z_ref, o_ref):
    n = pl.program_id(0)
    s2 = stdn_ref[n] * stdn_ref[n]
    o_ref[...] = x_ref[...] - s2 * z_ref[...]


def _flat_tiling(lout, halo):
    halo_pad = _round_up(halo, 128)
    tl = 2 * halo_pad
    nt = pl.cdiv(lout, tl)
    lf = nt * tl + halo_pad
    return tl, nt, lf, halo_pad


def _conv1_rbf(xpad, w, rbf_w, centers, prec, lb, ub, kh, kw):
    N, _, Hi, Wi = xpad.shape
    F = w.shape[0]
    KK = kh * kw
    Ho = Hi - kh + 1
    lout = Ho * Wi
    halo = (kh - 1) * Wi + (kw - 1)
    tl, nt, lf, halo_pad = _flat_tiling(lout, halo)

    xflat = jnp.pad(xpad.reshape(N, 1, Hi * Wi), ((0, 0), (0, 0), (0, lf - Hi * Wi)))
    sp = jnp.sqrt(jnp.asarray(prec, jnp.float32))
    scal = jnp.stack([sp, sp * jnp.asarray(lb, jnp.float32),
                      sp * jnp.asarray(ub, jnp.float32)])
    kk_pad = _round_up(KK, 8)
    w_stack = jnp.pad(w.transpose(0, 2, 3, 1).reshape(F, KK), ((0, 0), (0, kk_pad - KK)))

    ratio = tl // halo_pad
    x_spec = pl.BlockSpec((1, 1, tl), lambda n, t: (n, 0, t))
    halo_spec = pl.BlockSpec((1, 1, halo_pad), lambda n, t: (n, 0, (t + 1) * ratio))
    smem = pl.BlockSpec(memory_space=pltpu.MemorySpace.SMEM)
    out = pl.pallas_call(
        functools.partial(_conv1_rbf_kernel, kh=kh, kw=kw, wrow=Wi,
                          chunk=min(RBF_CHUNK, tl)),
        out_shape=jax.ShapeDtypeStruct((N, F, nt * tl), jnp.bfloat16),
        grid=(N, nt),
        in_specs=[smem, smem,
                  pl.BlockSpec((F, kk_pad), lambda n, t: (0, 0)),
                  pl.BlockSpec((F, centers.shape[0]), lambda n, t: (0, 0)),
                  x_spec, halo_spec],
        out_specs=pl.BlockSpec((1, F, tl), lambda n, t: (n, 0, t)),
        scratch_shapes=[pltpu.VMEM((1, tl + halo_pad), jnp.float32),
                        pltpu.VMEM((kk_pad, tl), jnp.float32),
                        pltpu.VMEM((F, tl), jnp.float32)],
        compiler_params=pltpu.CompilerParams(
            dimension_semantics=("parallel", "parallel"),
            vmem_limit_bytes=_VMEM_LIMIT),
    )(scal, sp * centers.astype(jnp.float32), w_stack,
      rbf_w.astype(jnp.float32), xflat, xflat)
    return out[:, :, :lout].reshape(N, F, Ho, Wi)[:, :, :, :Wi - kw + 1]


def _convT(v, w, kh, kw):
    N, F, H, W = v.shape
    Hi, Wi = H + 2 * (kh - 1), W + 2 * (kw - 1)
    Ho, Wo = Hi - kh + 1, Wi - kw + 1
    KK = kh * kw
    lout = Ho * Wi
    halo = (kh - 1) * Wi + (kw - 1)
    tl, nt, lf, halo_pad = _flat_tiling(lout, halo)

    vzp = jnp.pad(v, ((0, 0), (0, 0), (kh - 1, kh - 1), (kw - 1, kw - 1)))
    vflat = jnp.pad(vzp.reshape(N, F, Hi * Wi), ((0, 0), (0, 0), (0, lf - Hi * Wi)))

    wt = jnp.flip(w, (-2, -1)).reshape(F, KK).transpose(1, 0)
    kk_pad = _round_up(KK, 8)
    wt = jnp.pad(wt, ((0, kk_pad - KK), (0, 0))).astype(v.dtype)

    ratio = tl // halo_pad
    x_spec = pl.BlockSpec((1, F, tl), lambda n, t: (n, 0, t))
    halo_spec = pl.BlockSpec((1, F, halo_pad), lambda n, t: (n, 0, (t + 1) * ratio))
    out = pl.pallas_call(
        functools.partial(_convT_kernel, kh=kh, kw=kw, wrow=Wi),
        out_shape=jax.ShapeDtypeStruct((N, 1, nt * tl), jnp.float32),
        grid=(N, nt),
        in_specs=[pl.BlockSpec((kk_pad, F), lambda n, t: (0, 0)),
                  x_spec, halo_spec],
        out_specs=pl.BlockSpec((1, 1, tl), lambda n, t: (n, 0, t)),
        scratch_shapes=[pltpu.VMEM((F, tl + halo_pad), v.dtype),
                        pltpu.VMEM((kk_pad, tl + halo_pad), jnp.float32)],
        compiler_params=pltpu.CompilerParams(
            dimension_semantics=("parallel", "parallel"),
            vmem_limit_bytes=_VMEM_LIMIT),
    )(wt, vflat, vflat)
    return out[:, :, :lout].reshape(N, 1, Ho, Wi)[:, :, :, :Wo]


def _sym_pad_adjoint(z, pad):
    pt, pb, pL, pr = pad
    H2, W2 = z.shape[2], z.shape[3]
    core = z[:, :, pt:H2 - pb, :]
    if pt:
        core = core.at[:, :, :pt, :].add(z[:, :, :pt, :][:, :, ::-1, :])
    if pb:
        core = core.at[:, :, -pb:, :].add(z[:, :, H2 - pb:, :][:, :, ::-1, :])
    z = core
    core = z[:, :, :, pL:W2 - pr]
    if pL:
        core = core.at[:, :, :, :pL].add(z[:, :, :, :pL][:, :, :, ::-1])
    if pr:
        core = core.at[:, :, :, -pr:].add(z[:, :, :, W2 - pr:][:, :, :, ::-1])
    return core


def _residual(x, z, stdn, lanes=512, row_block=128):
    N, C, H, W = x.shape
    chw = C * H * W
    lanes = min(lanes, chw)
    rows = chw // lanes
    row_block = min(row_block, rows)
    x3 = x.reshape(N, rows, lanes)
    z3 = z.reshape(N, rows, lanes)
    blk = pl.BlockSpec((1, row_block, lanes), lambda n, r: (n, r, 0))
    smem = pl.BlockSpec(memory_space=pltpu.MemorySpace.SMEM)
    out = pl.pallas_call(
        _residual_kernel,
        out_shape=jax.ShapeDtypeStruct((N, rows, lanes), jnp.float32),
        grid=(N, rows // row_block),
        in_specs=[smem, blk, blk],
        out_specs=blk,
        compiler_params=pltpu.CompilerParams(
            dimension_semantics=("parallel", "parallel"),
            vmem_limit_bytes=_VMEM_LIMIT),
    )(stdn.astype(jnp.float32), x3, z3)
    return out.reshape(N, C, H, W)


def kernel(x, stdn, rbf_data, conv_weights, scale_f, alpha_prox, rbf_weights, rbf_centers):
    del rbf_data, alpha_prox
    kh = kw = 5
    M = rbf_centers.shape[0]
    lb, ub = -100.0, 100.0
    delta = (ub - lb) / (M - 1)
    prec = float(1.0 / (2.0 * delta * delta))
    pad = (kh // 2, kh // 2, kw // 2, kw // 2)

    w = conv_weights - conv_weights.mean(axis=(1, 2, 3), keepdims=True)
    nrm = jnp.sqrt(jnp.sum(w * w, axis=(1, 2, 3), keepdims=True)) + 1e-12
    w = (w / nrm) * scale_f[:, None, None, None]
    w = w.astype(jnp.float32)

    xpad = jnp.pad(x, ((0, 0), (0, 0), (pad[0], pad[1]), (pad[2], pad[3])),
                   mode="symmetric")
    v = _conv1_rbf(xpad, w, rbf_weights, rbf_centers, prec, lb, ub, kh, kw)
    zfull = _convT(v, w, kh, kw)
    z = _sym_pad_adjoint(zfull, pad)
    return _residual(x, z, stdn)


# speedup vs baseline: 1.8364x; 1.1046x over previous
import functools

import jax
import jax.numpy as jnp
from jax.experimental import pallas as pl
from jax.experimental.pallas import tpu as pltpu

RBF_CHUNK = 256
_VMEM_LIMIT = 48 * 1024 * 1024


def _round_up(a, b):
    return -(-a // b) * b


def _conv1_rbf_kernel(scal_ref, cent_ref, w_ref, rbfw_ref, x_ref, xh_ref,
                      o_ref, xbuf_ref, slab_ref, v_ref, *, kh, kw, wrow, chunk):
    tl = o_ref.shape[2]
    kk = kh * kw
    xbuf_ref[0, :tl] = x_ref[0, 0]
    xbuf_ref[0, tl:] = xh_ref[0, 0]
    for t in range(kk):
        i, j = divmod(t, kw)
        s = i * wrow + j
        slab_ref[t, :] = xbuf_ref[0, s:s + tl]
    slab_ref[kk:, :] = jnp.zeros((slab_ref.shape[0] - kk, tl), jnp.float32)
    v_ref[...] = jnp.dot(w_ref[...], slab_ref[...],
                         preferred_element_type=jnp.float32)
    sp = scal_ref[0]
    lb_s = scal_ref[1]
    ub_s = scal_ref[2]
    n_out = o_ref.shape[1]
    n_cent = cent_ref.shape[0]
    for c0 in range(0, tl, chunk):
        cw = min(chunk, tl - c0)
        u = jnp.clip(v_ref[:, c0:c0 + cw] * sp, lb_s, ub_s).astype(jnp.bfloat16)
        res = jnp.zeros((n_out, cw), jnp.bfloat16)
        for m in range(n_cent):
            d = u - cent_ref[m].astype(jnp.bfloat16)
            res = res + rbfw_ref[:, m:m + 1] * jnp.exp(-(d * d))
        o_ref[0, :, c0:c0 + cw] = res.astype(o_ref.dtype)


def _convT_kernel(w_ref, x_ref, xh_ref, o_ref, xbuf_ref, g_ref, *, kh, kw, wrow):
    tl = o_ref.shape[2]
    xbuf_ref[:, :tl] = x_ref[0]
    xbuf_ref[:, tl:] = xh_ref[0]
    g_ref[...] = jnp.dot(w_ref[...], xbuf_ref[...],
                         preferred_element_type=jnp.float32)
    acc = jnp.zeros((1, tl), jnp.float32)
    for t in range(kh * kw):
        i, j = divmod(t, kw)
        s = i * wrow + j
        acc = acc + g_ref[t:t + 1, s:s + tl]
    o_ref[0] = acc


def _residual_kernel(stdn_ref, x_ref, z_ref, o_ref):
    n = pl.program_id(0)
    s2 = stdn_ref[n] * stdn_ref[n]
    o_ref[...] = x_ref[...] - s2 * z_ref[...]


def _flat_tiling(lout, halo):
    halo_pad = _round_up(halo, 128)
    tl = 2 * halo_pad
    nt = pl.cdiv(lout, tl)
    lf = nt * tl + halo_pad
    return tl, nt, lf, halo_pad


def _conv1_rbf(xpad, w, rbf_w, centers, prec, lb, ub, kh, kw):
    N, _, Hi, Wi = xpad.shape
    F = w.shape[0]
    KK = kh * kw
    Ho = Hi - kh + 1
    lout = Ho * Wi
    halo = (kh - 1) * Wi + (kw - 1)
    tl, nt, lf, halo_pad = _flat_tiling(lout, halo)

    xflat = jnp.pad(xpad.reshape(N, 1, Hi * Wi), ((0, 0), (0, 0), (0, lf - Hi * Wi)))
    sp = jnp.sqrt(jnp.asarray(prec, jnp.float32))
    scal = jnp.stack([sp, sp * jnp.asarray(lb, jnp.float32),
                      sp * jnp.asarray(ub, jnp.float32)])
    kk_pad = _round_up(KK, 8)
    w_stack = jnp.pad(w.transpose(0, 2, 3, 1).reshape(F, KK), ((0, 0), (0, kk_pad - KK)))

    ratio = tl // halo_pad
    x_spec = pl.BlockSpec((1, 1, tl), lambda n, t: (n, 0, t))
    halo_spec = pl.BlockSpec((1, 1, halo_pad), lambda n, t: (n, 0, (t + 1) * ratio))
    smem = pl.BlockSpec(memory_space=pltpu.MemorySpace.SMEM)
    out = pl.pallas_call(
        functools.partial(_conv1_rbf_kernel, kh=kh, kw=kw, wrow=Wi,
                          chunk=min(RBF_CHUNK, tl)),
        out_shape=jax.ShapeDtypeStruct((N, F, nt * tl), jnp.bfloat16),
        grid=(N, nt),
        in_specs=[smem, smem,
                  pl.BlockSpec((F, kk_pad), lambda n, t: (0, 0)),
                  pl.BlockSpec((F, centers.shape[0]), lambda n, t: (0, 0)),
                  x_spec, halo_spec],
        out_specs=pl.BlockSpec((1, F, tl), lambda n, t: (n, 0, t)),
        scratch_shapes=[pltpu.VMEM((1, tl + halo_pad), jnp.float32),
                        pltpu.VMEM((kk_pad, tl), jnp.float32),
                        pltpu.VMEM((F, tl), jnp.float32)],
        compiler_params=pltpu.CompilerParams(
            dimension_semantics=("parallel", "parallel"),
            vmem_limit_bytes=_VMEM_LIMIT),
    )(scal, sp * centers.astype(jnp.float32), w_stack,
      rbf_w.astype(jnp.bfloat16), xflat, xflat)
    return out[:, :, :lout].reshape(N, F, Ho, Wi)[:, :, :, :Wi - kw + 1]


def _convT(v, w, kh, kw):
    N, F, H, W = v.shape
    Hi, Wi = H + 2 * (kh - 1), W + 2 * (kw - 1)
    Ho, Wo = Hi - kh + 1, Wi - kw + 1
    KK = kh * kw
    lout = Ho * Wi
    halo = (kh - 1) * Wi + (kw - 1)
    tl, nt, lf, halo_pad = _flat_tiling(lout, halo)

    vzp = jnp.pad(v, ((0, 0), (0, 0), (kh - 1, kh - 1), (kw - 1, kw - 1)))
    vflat = jnp.pad(vzp.reshape(N, F, Hi * Wi), ((0, 0), (0, 0), (0, lf - Hi * Wi)))

    wt = jnp.flip(w, (-2, -1)).reshape(F, KK).transpose(1, 0)
    kk_pad = _round_up(KK, 8)
    wt = jnp.pad(wt, ((0, kk_pad - KK), (0, 0))).astype(v.dtype)

    ratio = tl // halo_pad
    x_spec = pl.BlockSpec((1, F, tl), lambda n, t: (n, 0, t))
    halo_spec = pl.BlockSpec((1, F, halo_pad), lambda n, t: (n, 0, (t + 1) * ratio))
    out = pl.pallas_call(
        functools.partial(_convT_kernel, kh=kh, kw=kw, wrow=Wi),
        out_shape=jax.ShapeDtypeStruct((N, 1, nt * tl), jnp.float32),
        grid=(N, nt),
        in_specs=[pl.BlockSpec((kk_pad, F), lambda n, t: (0, 0)),
                  x_spec, halo_spec],
        out_specs=pl.BlockSpec((1, 1, tl), lambda n, t: (n, 0, t)),
        scratch_shapes=[pltpu.VMEM((F, tl + halo_pad), v.dtype),
                        pltpu.VMEM((kk_pad, tl + halo_pad), jnp.float32)],
        compiler_params=pltpu.CompilerParams(
            dimension_semantics=("parallel", "parallel"),
            vmem_limit_bytes=_VMEM_LIMIT),
    )(wt, vflat, vflat)
    return out[:, :, :lout].reshape(N, 1, Ho, Wi)[:, :, :, :Wo]


def _sym_pad_adjoint(z, pad):
    pt, pb, pL, pr = pad
    H2, W2 = z.shape[2], z.shape[3]
    core = z[:, :, pt:H2 - pb, :]
    if pt:
        core = core.at[:, :, :pt, :].add(z[:, :, :pt, :][:, :, ::-1, :])
    if pb:
        core = core.at[:, :, -pb:, :].add(z[:, :, H2 - pb:, :][:, :, ::-1, :])
    z = core
    core = z[:, :, :, pL:W2 - pr]
    if pL:
        core = core.at[:, :, :, :pL].add(z[:, :, :, :pL][:, :, :, ::-1])
    if pr:
        core = core.at[:, :, :, -pr:].add(z[:, :, :, W2 - pr:][:, :, :, ::-1])
    return core


def _residual(x, z, stdn, lanes=512, row_block=128):
    N, C, H, W = x.shape
    chw = C * H * W
    lanes = min(lanes, chw)
    rows = chw // lanes
    row_block = min(row_block, rows)
    x3 = x.reshape(N, rows, lanes)
    z3 = z.reshape(N, rows, lanes)
    blk = pl.BlockSpec((1, row_block, lanes), lambda n, r: (n, r, 0))
    smem = pl.BlockSpec(memory_space=pltpu.MemorySpace.SMEM)
    out = pl.pallas_call(
        _residual_kernel,
        out_shape=jax.ShapeDtypeStruct((N, rows, lanes), jnp.float32),
        grid=(N, rows // row_block),
        in_specs=[smem, blk, blk],
        out_specs=blk,
        compiler_params=pltpu.CompilerParams(
            dimension_semantics=("parallel", "parallel"),
            vmem_limit_bytes=_VMEM_LIMIT),
    )(stdn.astype(jnp.float32), x3, z3)
    return out.reshape(N, C, H, W)


def kernel(x, stdn, rbf_data, conv_weights, scale_f, alpha_prox, rbf_weights, rbf_centers):
    del rbf_data, alpha_prox
    kh = kw = 5
    M = rbf_centers.shape[0]
    lb, ub = -100.0, 100.0
    delta = (ub - lb) / (M - 1)
    prec = float(1.0 / (2.0 * delta * delta))
    pad = (kh // 2, kh // 2, kw // 2, kw // 2)

    w = conv_weights - conv_weights.mean(axis=(1, 2, 3), keepdims=True)
    nrm = jnp.sqrt(jnp.sum(w * w, axis=(1, 2, 3), keepdims=True)) + 1e-12
    w = (w / nrm) * scale_f[:, None, None, None]
    w = w.astype(jnp.float32)

    xpad = jnp.pad(x, ((0, 0), (0, 0), (pad[0], pad[1]), (pad[2], pad[3])),
                   mode="symmetric")
    v = _conv1_rbf(xpad, w, rbf_weights, rbf_centers, prec, lb, ub, kh, kw)
    zfull = _convT(v, w, kh, kw)
    z = _sym_pad_adjoint(zfull, pad)
    return _residual(x, z, stdn)


# speedup vs baseline: 1.9556x; 1.0649x over previous
import functools

import jax
import jax.numpy as jnp
from jax.experimental import pallas as pl
from jax.experimental.pallas import tpu as pltpu

RBF_CHUNK = 512
_VMEM_LIMIT = 48 * 1024 * 1024


def _round_up(a, b):
    return -(-a // b) * b


def _conv1_rbf_kernel(scal_ref, cent_ref, w_ref, rbfw_ref, x_ref, xh_ref,
                      o_ref, xbuf_ref, slab_ref, v_ref, *, kh, kw, wrow, chunk):
    tl = o_ref.shape[2]
    kk = kh * kw
    xbuf_ref[0, :tl] = x_ref[0, 0]
    xbuf_ref[0, tl:] = xh_ref[0, 0]
    for t in range(kk):
        i, j = divmod(t, kw)
        s = i * wrow + j
        slab_ref[t, :] = xbuf_ref[0, s:s + tl]
    slab_ref[kk:, :] = jnp.zeros((slab_ref.shape[0] - kk, tl), jnp.float32)
    v_ref[...] = jnp.dot(w_ref[...], slab_ref[...],
                         preferred_element_type=jnp.float32)
    sp = scal_ref[0]
    lb_s = scal_ref[1]
    ub_s = scal_ref[2]
    n_out = o_ref.shape[1]
    n_cent = cent_ref.shape[0]
    for c0 in range(0, tl, chunk):
        cw = min(chunk, tl - c0)
        u = jnp.clip(v_ref[:, c0:c0 + cw] * sp, lb_s, ub_s).astype(jnp.bfloat16)
        res = jnp.zeros((n_out, cw), jnp.bfloat16)
        for m in range(n_cent):
            d = u - cent_ref[m].astype(jnp.bfloat16)
            res = res + rbfw_ref[:, m:m + 1] * jnp.exp(-(d * d))
        o_ref[0, :, c0:c0 + cw] = res.astype(o_ref.dtype)


def _convT_kernel(w_ref, x_ref, xh_ref, o_ref, xbuf_ref, g_ref, *, kh, kw, wrow):
    tl = o_ref.shape[2]
    xbuf_ref[:, :tl] = x_ref[0]
    xbuf_ref[:, tl:] = xh_ref[0]
    g_ref[...] = jnp.dot(w_ref[...], xbuf_ref[...],
                         preferred_element_type=jnp.float32)
    acc = jnp.zeros((1, tl), jnp.float32)
    for t in range(kh * kw):
        i, j = divmod(t, kw)
        s = i * wrow + j
        acc = acc + g_ref[t:t + 1, s:s + tl]
    o_ref[0] = acc


def _residual_kernel(stdn_ref, x_ref, z_ref, o_ref):
    n = pl.program_id(0)
    s2 = stdn_ref[n] * stdn_ref[n]
    o_ref[...] = x_ref[...] - s2 * z_ref[...]


def _flat_tiling(lout, halo):
    halo_pad = _round_up(halo, 128)
    tl = 2 * halo_pad
    nt = pl.cdiv(lout, tl)
    lf = nt * tl + halo_pad
    return tl, nt, lf, halo_pad


def _conv1_rbf(xpad, w, rbf_w, centers, prec, lb, ub, kh, kw):
    N, _, Hi, Wi = xpad.shape
    F = w.shape[0]
    KK = kh * kw
    Ho = Hi - kh + 1
    lout = Ho * Wi
    halo = (kh - 1) * Wi + (kw - 1)
    tl, nt, lf, halo_pad = _flat_tiling(lout, halo)

    xflat = jnp.pad(xpad.reshape(N, 1, Hi * Wi), ((0, 0), (0, 0), (0, lf - Hi * Wi)))
    sp = jnp.sqrt(jnp.asarray(prec, jnp.float32))
    scal = jnp.stack([sp, sp * jnp.asarray(lb, jnp.float32),
                      sp * jnp.asarray(ub, jnp.float32)])
    kk_pad = _round_up(KK, 8)
    w_stack = jnp.pad(w.transpose(0, 2, 3, 1).reshape(F, KK), ((0, 0), (0, kk_pad - KK)))

    ratio = tl // halo_pad
    x_spec = pl.BlockSpec((1, 1, tl), lambda n, t: (n, 0, t))
    halo_spec = pl.BlockSpec((1, 1, halo_pad), lambda n, t: (n, 0, (t + 1) * ratio))
    smem = pl.BlockSpec(memory_space=pltpu.MemorySpace.SMEM)
    out = pl.pallas_call(
        functools.partial(_conv1_rbf_kernel, kh=kh, kw=kw, wrow=Wi,
                          chunk=min(RBF_CHUNK, tl)),
        out_shape=jax.ShapeDtypeStruct((N, F, nt * tl), jnp.bfloat16),
        grid=(N, nt),
        in_specs=[smem, smem,
                  pl.BlockSpec((F, kk_pad), lambda n, t: (0, 0)),
                  pl.BlockSpec((F, centers.shape[0]), lambda n, t: (0, 0)),
                  x_spec, halo_spec],
        out_specs=pl.BlockSpec((1, F, tl), lambda n, t: (n, 0, t)),
        scratch_shapes=[pltpu.VMEM((1, tl + halo_pad), jnp.float32),
                        pltpu.VMEM((kk_pad, tl), jnp.float32),
                        pltpu.VMEM((F, tl), jnp.float32)],
        compiler_params=pltpu.CompilerParams(
            dimension_semantics=("parallel", "parallel"),
            vmem_limit_bytes=_VMEM_LIMIT),
    )(scal, sp * centers.astype(jnp.float32), w_stack,
      rbf_w.astype(jnp.bfloat16), xflat, xflat)
    return out[:, :, :lout].reshape(N, F, Ho, Wi)[:, :, :, :Wi - kw + 1]


def _convT(v, w, kh, kw):
    N, F, H, W = v.shape
    Hi, Wi = H + 2 * (kh - 1), W + 2 * (kw - 1)
    Ho, Wo = Hi - kh + 1, Wi - kw + 1
    KK = kh * kw
    lout = Ho * Wi
    halo = (kh - 1) * Wi + (kw - 1)
    tl, nt, lf, halo_pad = _flat_tiling(lout, halo)

    vzp = jnp.pad(v, ((0, 0), (0, 0), (kh - 1, kh - 1), (kw - 1, kw - 1)))
    vflat = jnp.pad(vzp.reshape(N, F, Hi * Wi), ((0, 0), (0, 0), (0, lf - Hi * Wi)))

    wt = jnp.flip(w, (-2, -1)).reshape(F, KK).transpose(1, 0)
    kk_pad = _round_up(KK, 8)
    wt = jnp.pad(wt, ((0, kk_pad - KK), (0, 0))).astype(v.dtype)

    ratio = tl // halo_pad
    x_spec = pl.BlockSpec((1, F, tl), lambda n, t: (n, 0, t))
    halo_spec = pl.BlockSpec((1, F, halo_pad), lambda n, t: (n, 0, (t + 1) * ratio))
    out = pl.pallas_call(
        functools.partial(_convT_kernel, kh=kh, kw=kw, wrow=Wi),
        out_shape=jax.ShapeDtypeStruct((N, 1, nt * tl), jnp.float32),
        grid=(N, nt),
        in_specs=[pl.BlockSpec((kk_pad, F), lambda n, t: (0, 0)),
                  x_spec, halo_spec],
        out_specs=pl.BlockSpec((1, 1, tl), lambda n, t: (n, 0, t)),
        scratch_shapes=[pltpu.VMEM((F, tl + halo_pad), v.dtype),
                        pltpu.VMEM((kk_pad, tl + halo_pad), jnp.float32)],
        compiler_params=pltpu.CompilerParams(
            dimension_semantics=("parallel", "parallel"),
            vmem_limit_bytes=_VMEM_LIMIT),
    )(wt, vflat, vflat)
    return out[:, :, :lout].reshape(N, 1, Ho, Wi)[:, :, :, :Wo]


def _sym_pad_adjoint(z, pad):
    pt, pb, pL, pr = pad
    H2, W2 = z.shape[2], z.shape[3]
    core = z[:, :, pt:H2 - pb, :]
    if pt:
        core = core.at[:, :, :pt, :].add(z[:, :, :pt, :][:, :, ::-1, :])
    if pb:
        core = core.at[:, :, -pb:, :].add(z[:, :, H2 - pb:, :][:, :, ::-1, :])
    z = core
    core = z[:, :, :, pL:W2 - pr]
    if pL:
        core = core.at[:, :, :, :pL].add(z[:, :, :, :pL][:, :, :, ::-1])
    if pr:
        core = core.at[:, :, :, -pr:].add(z[:, :, :, W2 - pr:][:, :, :, ::-1])
    return core


def _residual(x, z, stdn, lanes=512, row_block=128):
    N, C, H, W = x.shape
    chw = C * H * W
    lanes = min(lanes, chw)
    rows = chw // lanes
    row_block = min(row_block, rows)
    x3 = x.reshape(N, rows, lanes)
    z3 = z.reshape(N, rows, lanes)
    blk = pl.BlockSpec((1, row_block, lanes), lambda n, r: (n, r, 0))
    smem = pl.BlockSpec(memory_space=pltpu.MemorySpace.SMEM)
    out = pl.pallas_call(
        _residual_kernel,
        out_shape=jax.ShapeDtypeStruct((N, rows, lanes), jnp.float32),
        grid=(N, rows // row_block),
        in_specs=[smem, blk, blk],
        out_specs=blk,
        compiler_params=pltpu.CompilerParams(
            dimension_semantics=("parallel", "parallel"),
            vmem_limit_bytes=_VMEM_LIMIT),
    )(stdn.astype(jnp.float32), x3, z3)
    return out.reshape(N, C, H, W)


def kernel(x, stdn, rbf_data, conv_weights, scale_f, alpha_prox, rbf_weights, rbf_centers):
    del rbf_data, alpha_prox
    kh = kw = 5
    M = rbf_centers.shape[0]
    lb, ub = -100.0, 100.0
    delta = (ub - lb) / (M - 1)
    prec = float(1.0 / (2.0 * delta * delta))
    pad = (kh // 2, kh // 2, kw // 2, kw // 2)

    w = conv_weights - conv_weights.mean(axis=(1, 2, 3), keepdims=True)
    nrm = jnp.sqrt(jnp.sum(w * w, axis=(1, 2, 3), keepdims=True)) + 1e-12
    w = (w / nrm) * scale_f[:, None, None, None]
    w = w.astype(jnp.float32)

    xpad = jnp.pad(x, ((0, 0), (0, 0), (pad[0], pad[1]), (pad[2], pad[3])),
                   mode="symmetric")
    v = _conv1_rbf(xpad, w, rbf_weights, rbf_centers, prec, lb, ub, kh, kw)
    zfull = _convT(v, w, kh, kw)
    z = _sym_pad_adjoint(zfull, pad)
    return _residual(x, z, stdn)
